# Initial kernel scaffold; baseline (speedup 1.0000x reference)
import jax
import jax.numpy as jnp
from jax.experimental import pallas as pl


def kernel(x_0, x_1, neighborhood_0_to_0, att):
    raise NotImplementedError("write your pallas kernel here")


# speedup vs baseline: 4.8072x; 4.8072x over previous
import functools

import jax
import jax.numpy as jnp
from jax import lax
from jax.experimental import pallas as pl
from jax.experimental.pallas import tpu as pltpu
from jax.experimental.pallas import tpu_sc as plsc

N_NODES = 10000
N_EDGES = 320000
D_FEAT = 128
K_HEADS = 3
D_EDGE = 16
TBL_W = 8

NC = 2
NS = 16
NW = NC * NS
E_PER_W = N_EDGES // NW
CHUNK = 2000
N_CHUNKS = E_PER_W // CHUNK
LANES = 16


def _project_body(x_ref, w_ref, out_ref):
    out_ref[...] = lax.dot_general(
        x_ref[...], w_ref[...], (((1,), (0,)), ((), ())),
        preferred_element_type=jnp.float32,
        precision=lax.Precision.HIGHEST)


def _project(x_0, w):
    return pl.pallas_call(
        _project_body,
        out_shape=jax.ShapeDtypeStruct((N_NODES, TBL_W), jnp.float32),
    )(x_0, w)


def _lift_body(table_hbm, src_hbm, tgt_hbm, out_hbm, table_v, src_v, tgt_v, out_v):
    wid = lax.axis_index("s") * NC + lax.axis_index("c")
    base = wid * E_PER_W
    pltpu.sync_copy(table_hbm, table_v)

    for c in range(N_CHUNKS):
        row0 = base + c * CHUNK
        pltpu.sync_copy(src_hbm.at[pl.ds(row0, CHUNK)], src_v)
        pltpu.sync_copy(tgt_hbm.at[pl.ds(row0, CHUNK)], tgt_v)

        def body(i, carry):
            s_idx = src_v[pl.ds(i * LANES, LANES)] * TBL_W
            t_idx = tgt_v[pl.ds(i * LANES, LANES)] * TBL_W
            e = i * (LANES * K_HEADS) + lax.iota(jnp.int32, LANES) * K_HEADS
            for k in range(K_HEADS):
                a = plsc.load_gather(table_v, [s_idx + k])
                b = plsc.load_gather(table_v, [t_idx + (K_HEADS + k)])
                h = jnp.maximum(a + b, 0.0)
                plsc.store_scatter(out_v, [e + k], h)
            return carry

        lax.fori_loop(0, CHUNK // LANES, body, 0)
        pltpu.sync_copy(out_v, out_hbm.at[pl.ds(row0 * K_HEADS, CHUNK * K_HEADS)])


def _lift(table, src, tgt):
    return pl.kernel(
        _lift_body,
        out_type=jax.ShapeDtypeStruct((N_EDGES * K_HEADS,), jnp.float32),
        mesh=plsc.VectorSubcoreMesh(core_axis_name="c", subcore_axis_name="s"),
        compiler_params=pltpu.CompilerParams(needs_layout_passes=False),
        scratch_types=[
            pltpu.VMEM((N_NODES * TBL_W,), jnp.float32),
            pltpu.VMEM((CHUNK,), jnp.int32),
            pltpu.VMEM((CHUNK,), jnp.int32),
            pltpu.VMEM((CHUNK * K_HEADS,), jnp.float32),
        ],
    )(table, src, tgt)


def kernel(x_0, x_1, neighborhood_0_to_0, att):
    idx = neighborhood_0_to_0.astype(jnp.int32)
    src, tgt = idx[0], idx[1]
    a = att[:, :, 0]
    w_s = a[:, :D_FEAT].T
    w_t = a[:, D_FEAT:].T
    w = jnp.concatenate(
        [w_s, w_t, jnp.zeros((D_FEAT, TBL_W - 2 * K_HEADS), jnp.float32)],
        axis=1)
    table = _project(x_0, w)
    heads = _lift(table.reshape(-1), src, tgt).reshape(N_EDGES, K_HEADS)
    return jnp.concatenate([heads, x_1], axis=1)
